# Initial kernel scaffold; baseline (speedup 1.0000x reference)
import jax
import jax.numpy as jnp
from jax.experimental import pallas as pl


def kernel(z, W1, b1, W2, b2, W3, b3, a_raw, b_param):
    raise NotImplementedError("write your pallas kernel here")


# speedup vs baseline: 450.3665x; 450.3665x over previous
import jax
import jax.numpy as jnp
from jax.experimental import pallas as pl
from jax.experimental.pallas import tpu as pltpu

NPTS = 400
ZMIN, ZMAX = -3.0, 3.0
CLIP = 1.0
HID = 64
PPAD = 512
DT = (ZMAX - ZMIN) / (NPTS - 1)
INV_DT = 1.0 / DT

B = 8388608
COLS = 128
ROWS = B // COLS
BR = 4096
NBLK = ROWS // BR
NCORE = 2
NB = NBLK // NCORE


def _tables_kernel(w1_ref, b1_ref, w2_ref, b2_ref, w3_ref, b3_ref, ab_ref):
    tf = ZMIN + DT * jax.lax.broadcasted_iota(
        jnp.int32, (PPAD, HID), 0).astype(jnp.float32)
    h = jnp.tanh(tf * w1_ref[...] + b1_ref[...])
    h = jnp.tanh(
        jax.lax.dot_general(h, w2_ref[...], (((1,), (1,)), ((), ())),
                            preferred_element_type=jnp.float32)
        + b2_ref[...])
    g = jnp.sum(h * w3_ref[...], axis=1, keepdims=True) + b3_ref[...]
    w = jnp.exp(jnp.clip(g, -CLIP, CLIP))
    ir = jax.lax.broadcasted_iota(jnp.int32, (PPAD, PPAD), 0)
    jc = jax.lax.broadcasted_iota(jnp.int32, (PPAD, PPAD), 1)
    coeff = ((jc < ir).astype(jnp.float32)
             + ((jc >= 1) & (jc <= ir)).astype(jnp.float32)) * (0.5 * DT)
    F = jax.lax.dot_general(coeff, w, (((1,), (0,)), ((), ())),
                            preferred_element_type=jnp.float32)
    tcol = ZMIN + DT * jax.lax.broadcasted_iota(
        jnp.int32, (PPAD, 1), 0).astype(jnp.float32)
    alpha = F - tcol * w
    ab_ref[...] = jnp.concatenate([alpha, w], axis=1)


def _interp(tab_ref, z):
    n = z.shape[0]
    pos = (z - ZMIN) * INV_DT
    kf = jnp.clip(jnp.floor(pos), 0.0, float(NPTS - 2))
    kf = jnp.where(z > ZMAX, float(NPTS - 1), kf)
    k = kf.astype(jnp.int32)
    lo = jnp.bitwise_and(k, 127)
    hi = jnp.right_shift(k, 7)

    def gather(row):
        t = jnp.broadcast_to(tab_ref[row:row + 1, :], (n, COLS))
        return jnp.take_along_axis(t, lo, axis=1)

    a0, a1, a2, a3 = gather(0), gather(1), gather(2), gather(3)
    b0, b1, b2, b3 = gather(4), gather(5), gather(6), gather(7)
    low = hi < 2
    is0 = hi == 0
    is2 = hi == 2
    alpha = jnp.where(low, jnp.where(is0, a0, a1), jnp.where(is2, a2, a3))
    beta = jnp.where(low, jnp.where(is0, b0, b1), jnp.where(is2, b2, b3))
    return alpha + z * beta


def _reduce_kernel(tab_ref, z_ref, out_ref):
    i = pl.program_id(1)

    @pl.when(i == 0)
    def _():
        out_ref[...] = jnp.zeros_like(out_ref)

    fz = _interp(tab_ref, z_ref[...])
    out_ref[0, 0:1, :] += jnp.sum(fz, axis=0, keepdims=True)
    out_ref[0, 1:2, :] += jnp.sum(fz * fz, axis=0, keepdims=True)


def _apply_kernel(sc_ref, tab_ref, z_ref, o_ref):
    fz = _interp(tab_ref, z_ref[...])
    o_ref[...] = sc_ref[0] * fz + sc_ref[1]


def kernel(z, W1, b1, W2, b2, W3, b3, a_raw, b_param):
    zr = z.reshape(ROWS, COLS)

    ab = pl.pallas_call(
        _tables_kernel,
        out_shape=jax.ShapeDtypeStruct((PPAD, 2), jnp.float32),
        name="innv4_tables",
    )(W1.reshape(1, HID), b1.reshape(1, HID), W2, b2.reshape(1, HID),
      W3.reshape(1, HID), b3.reshape(1, 1))

    tab = jnp.concatenate(
        [ab[:, 0].reshape(4, COLS), ab[:, 1].reshape(4, COLS)], axis=0)

    part = pl.pallas_call(
        _reduce_kernel,
        out_shape=jax.ShapeDtypeStruct((NCORE, 2, COLS), jnp.float32),
        grid=(NCORE, NB),
        in_specs=[
            pl.BlockSpec((8, COLS), lambda c, i: (0, 0)),
            pl.BlockSpec((BR, COLS), lambda c, i: (c * NB + i, 0)),
        ],
        out_specs=pl.BlockSpec((1, 2, COLS), lambda c, i: (c, 0, 0)),
        compiler_params=pltpu.CompilerParams(
            dimension_semantics=("parallel", "arbitrary")),
        name="innv4_reduce",
    )(tab, zr)

    s1 = jnp.sum(part[:, 0, :])
    s2 = jnp.sum(part[:, 1, :])
    nb = jnp.float32(B)
    mu = s1 / nb
    var = (s2 - s1 * s1 / nb) / (nb - 1.0)
    sig = jnp.maximum(jnp.sqrt(jnp.maximum(var, 0.0)), 1e-6)
    a = jax.nn.softplus(a_raw) + 1e-3
    s = a / sig
    c = b_param - mu * s
    sc = jnp.stack([s, c]).astype(jnp.float32)

    out = pl.pallas_call(
        _apply_kernel,
        out_shape=jax.ShapeDtypeStruct((ROWS, COLS), jnp.float32),
        grid=(NCORE, NB),
        in_specs=[
            pl.BlockSpec(memory_space=pltpu.SMEM),
            pl.BlockSpec((8, COLS), lambda c, i: (0, 0)),
            pl.BlockSpec((BR, COLS), lambda c, i: (c * NB + i, 0)),
        ],
        out_specs=pl.BlockSpec((BR, COLS), lambda c, i: (c * NB + i, 0)),
        compiler_params=pltpu.CompilerParams(
            dimension_semantics=("parallel", "arbitrary")),
        name="innv4_apply",
    )(sc, tab, zr)

    return out.reshape(B, 1)


# speedup vs baseline: 837.8385x; 1.8603x over previous
import jax
import jax.numpy as jnp
from jax.experimental import pallas as pl
from jax.experimental.pallas import tpu as pltpu

NPTS = 400
ZMIN, ZMAX = -3.0, 3.0
CLIP = 1.0
HID = 64
PPAD = 512
DT = (ZMAX - ZMIN) / (NPTS - 1)
INV_DT = 1.0 / DT

B = 8388608
COLS = 128
ROWS = B // COLS
BR = 4096
NBLK = ROWS // BR
CH = 8


def _tables_kernel(w1_ref, b1_ref, w2_ref, b2_ref, w3_ref, b3_ref, ab_ref):
    tf = ZMIN + DT * jax.lax.broadcasted_iota(
        jnp.int32, (PPAD, HID), 0).astype(jnp.float32)
    h = jnp.tanh(tf * w1_ref[...] + b1_ref[...])
    h = jnp.tanh(
        jax.lax.dot_general(h, w2_ref[...], (((1,), (1,)), ((), ())),
                            preferred_element_type=jnp.float32)
        + b2_ref[...])
    g = jnp.sum(h * w3_ref[...], axis=1, keepdims=True) + b3_ref[...]
    w = jnp.exp(jnp.clip(g, -CLIP, CLIP))
    ir = jax.lax.broadcasted_iota(jnp.int32, (PPAD, PPAD), 0)
    jc = jax.lax.broadcasted_iota(jnp.int32, (PPAD, PPAD), 1)
    coeff = ((jc < ir).astype(jnp.float32)
             + ((jc >= 1) & (jc <= ir)).astype(jnp.float32)) * (0.5 * DT)
    F = jax.lax.dot_general(coeff, w, (((1,), (0,)), ((), ())),
                            preferred_element_type=jnp.float32)
    tcol = ZMIN + DT * jax.lax.broadcasted_iota(
        jnp.int32, (PPAD, 1), 0).astype(jnp.float32)
    alpha = F - tcol * w
    ab_ref[...] = jnp.concatenate([alpha, w], axis=1)


def _interp_chunk(trows, z):
    pos = (z - ZMIN) * INV_DT
    k = jnp.clip(pos, 0.0, float(NPTS - 2)).astype(jnp.int32)
    k = jnp.where(z > ZMAX, NPTS - 1, k)
    lo = jnp.bitwise_and(k, 127)
    hi = jnp.right_shift(k, 7)
    low = hi < 2
    is0 = hi == 0
    is2 = hi == 2

    def gather(row):
        return jnp.take_along_axis(trows[row], lo, axis=1)

    a01 = jnp.where(is0, gather(0), gather(1))
    a23 = jnp.where(is2, gather(2), gather(3))
    alpha = jnp.where(low, a01, a23)
    b01 = jnp.where(is0, gather(4), gather(5))
    b23 = jnp.where(is2, gather(6), gather(7))
    beta = jnp.where(low, b01, b23)
    return alpha + z * beta


def _reduce_kernel(tab_ref, z_ref, out_ref):
    i = pl.program_id(0)
    trows = [tab_ref[r] for r in range(8)]
    acc_s = jnp.zeros((CH, COLS), jnp.float32)
    acc_q = jnp.zeros((CH, COLS), jnp.float32)
    for j in range(BR // CH):
        zc = z_ref[pl.ds(j * CH, CH), :]
        fz = _interp_chunk(trows, zc)
        acc_s = acc_s + fz
        acc_q = acc_q + fz * fz

    @pl.when(i == 0)
    def _():
        out_ref[...] = jnp.zeros_like(out_ref)

    out_ref[0] += acc_s
    out_ref[1] += acc_q


def _apply_kernel(sc_ref, tab_ref, z_ref, o_ref):
    trows = [tab_ref[r] for r in range(8)]
    for j in range(BR // CH):
        zc = z_ref[pl.ds(j * CH, CH), :]
        fz = _interp_chunk(trows, zc)
        o_ref[pl.ds(j * CH, CH), :] = sc_ref[0] * fz + sc_ref[1]


def kernel(z, W1, b1, W2, b2, W3, b3, a_raw, b_param):
    zr = z.reshape(ROWS, COLS)

    ab = pl.pallas_call(
        _tables_kernel,
        out_shape=jax.ShapeDtypeStruct((PPAD, 2), jnp.float32),
        name="innv4_tables",
    )(W1.reshape(1, HID), b1.reshape(1, HID), W2, b2.reshape(1, HID),
      W3.reshape(1, HID), b3.reshape(1, 1))

    tab = jnp.concatenate(
        [ab[:, 0].reshape(4, COLS), ab[:, 1].reshape(4, COLS)], axis=0)
    tab8 = jnp.broadcast_to(tab[:, None, :], (8, CH, COLS))

    part = pl.pallas_call(
        _reduce_kernel,
        out_shape=jax.ShapeDtypeStruct((2, 8, COLS), jnp.float32),
        grid=(NBLK,),
        in_specs=[
            pl.BlockSpec((8, CH, COLS), lambda i: (0, 0, 0)),
            pl.BlockSpec((BR, COLS), lambda i: (i, 0)),
        ],
        out_specs=pl.BlockSpec((2, CH, COLS), lambda i: (0, 0, 0)),
        compiler_params=pltpu.CompilerParams(
            dimension_semantics=("arbitrary",)),
        name="innv4_reduce",
    )(tab8, zr)

    s1 = jnp.sum(part[0])
    s2 = jnp.sum(part[1])
    nb = jnp.float32(B)
    mu = s1 / nb
    var = (s2 - s1 * s1 / nb) / (nb - 1.0)
    sig = jnp.maximum(jnp.sqrt(jnp.maximum(var, 0.0)), 1e-6)
    a = jax.nn.softplus(a_raw) + 1e-3
    s = a / sig
    c = b_param - mu * s
    sc = jnp.stack([s, c]).astype(jnp.float32)

    out = pl.pallas_call(
        _apply_kernel,
        out_shape=jax.ShapeDtypeStruct((ROWS, COLS), jnp.float32),
        grid=(NBLK,),
        in_specs=[
            pl.BlockSpec(memory_space=pltpu.SMEM),
            pl.BlockSpec((8, CH, COLS), lambda i: (0, 0, 0)),
            pl.BlockSpec((BR, COLS), lambda i: (i, 0)),
        ],
        out_specs=pl.BlockSpec((BR, COLS), lambda i: (i, 0)),
        compiler_params=pltpu.CompilerParams(
            dimension_semantics=("arbitrary",)),
        name="innv4_apply",
    )(sc, tab8, zr)

    return out.reshape(B, 1)


# speedup vs baseline: 1204.5786x; 1.4377x over previous
import jax
import jax.numpy as jnp
from jax.experimental import pallas as pl
from jax.experimental.pallas import tpu as pltpu

NPTS = 400
ZMIN, ZMAX = -3.0, 3.0
CLIP = 1.0
HID = 64
PPAD = 512
DT = (ZMAX - ZMIN) / (NPTS - 1)
INV_DT = 1.0 / DT

B = 8388608
COLS = 128
ROWS = B // COLS
BR = 4096
NBLK = ROWS // BR
CH = 8


def _tables_kernel(w1_ref, b1_ref, w2_ref, b2_ref, w3_ref, b3_ref,
                   pk_ref, cst_ref):
    tf = ZMIN + DT * jax.lax.broadcasted_iota(
        jnp.int32, (PPAD, HID), 0).astype(jnp.float32)
    h = jnp.tanh(tf * w1_ref[...] + b1_ref[...])
    h = jnp.tanh(
        jax.lax.dot_general(h, w2_ref[...], (((1,), (1,)), ((), ())),
                            preferred_element_type=jnp.float32)
        + b2_ref[...])
    g = jnp.sum(h * w3_ref[...], axis=1, keepdims=True) + b3_ref[...]
    w = jnp.exp(jnp.clip(g, -CLIP, CLIP))
    ir = jax.lax.broadcasted_iota(jnp.int32, (PPAD, PPAD), 0)
    jc = jax.lax.broadcasted_iota(jnp.int32, (PPAD, PPAD), 1)
    coeff = ((jc < ir).astype(jnp.float32)
             + ((jc >= 1) & (jc <= ir)).astype(jnp.float32)) * (0.5 * DT)
    F = jax.lax.dot_general(coeff, w, (((1,), (0,)), ((), ())),
                            preferred_element_type=jnp.float32)
    icol = jax.lax.broadcasted_iota(jnp.int32, (PPAD, 1), 0)
    tcol = ZMIN + DT * icol.astype(jnp.float32)
    valid = icol < NPTS
    alpha = jnp.where(valid, F - tcol * w, 0.0)
    beta = jnp.where(valid, w, 1.0)
    amin = jnp.min(alpha, keepdims=True)
    arange = jnp.max(alpha, keepdims=True) - amin
    sa = jnp.maximum(arange * (1.0 / 65535.0), 1e-30)
    bmin = jnp.min(beta, keepdims=True)
    brange = jnp.max(beta, keepdims=True) - bmin
    sb = jnp.maximum(brange * (1.0 / 65535.0), 1e-30)
    qa = ((alpha - amin) * (1.0 / sa) + 0.5).astype(jnp.int32) - 32768
    qb = ((beta - bmin) * (1.0 / sb) + 0.5).astype(jnp.int32)
    pk_ref[...] = jnp.left_shift(qa, 16) | qb
    amin_adj = amin + 32768.0 * sa
    cst_ref[...] = jnp.concatenate([amin_adj, sa, bmin, sb], axis=1)


def _interp_chunk(trows, c_amin, c_sa, c_bmin, c_sb, z):
    pos = (z - ZMIN) * INV_DT
    k = jnp.clip(pos, 0.0, float(NPTS - 2)).astype(jnp.int32)
    k = jnp.where(z > ZMAX, NPTS - 1, k)
    lo = jnp.bitwise_and(k, 127)
    hi = jnp.right_shift(k, 7)
    low = hi < 2
    is0 = hi == 0
    is2 = hi == 2

    def gather(row):
        return jnp.take_along_axis(trows[row], lo, axis=1)

    v01 = jnp.where(is0, gather(0), gather(1))
    v23 = jnp.where(is2, gather(2), gather(3))
    v = jnp.where(low, v01, v23)
    qa = jnp.right_shift(v, 16).astype(jnp.float32)
    qb = jnp.bitwise_and(v, 0xFFFF).astype(jnp.float32)
    t2 = qa * c_sa + c_amin
    t1 = qb * c_sb + c_bmin
    return t1 * z + t2


def _reduce_kernel(cst_ref, tab_ref, z_ref, out_ref):
    i = pl.program_id(0)
    trows = [tab_ref[r] for r in range(4)]
    c_amin, c_sa, c_bmin, c_sb = (cst_ref[0], cst_ref[1],
                                  cst_ref[2], cst_ref[3])
    acc_s = jnp.zeros((CH, COLS), jnp.float32)
    acc_q = jnp.zeros((CH, COLS), jnp.float32)
    for j in range(BR // CH):
        zc = z_ref[pl.ds(j * CH, CH), :]
        fz = _interp_chunk(trows, c_amin, c_sa, c_bmin, c_sb, zc)
        acc_s = acc_s + fz
        acc_q = acc_q + fz * fz

    @pl.when(i == 0)
    def _():
        out_ref[...] = jnp.zeros_like(out_ref)

    out_ref[0] += acc_s
    out_ref[1] += acc_q


def _apply_kernel(cst_ref, tab_ref, z_ref, o_ref):
    trows = [tab_ref[r] for r in range(4)]
    c_amin, c_sa, c_bmin, c_sb = (cst_ref[0], cst_ref[1],
                                  cst_ref[2], cst_ref[3])
    for j in range(BR // CH):
        zc = z_ref[pl.ds(j * CH, CH), :]
        o_ref[pl.ds(j * CH, CH), :] = _interp_chunk(
            trows, c_amin, c_sa, c_bmin, c_sb, zc)


def kernel(z, W1, b1, W2, b2, W3, b3, a_raw, b_param):
    zr = z.reshape(ROWS, COLS)

    pk, cst = pl.pallas_call(
        _tables_kernel,
        out_shape=(jax.ShapeDtypeStruct((PPAD, 1), jnp.int32),
                   jax.ShapeDtypeStruct((1, 4), jnp.float32)),
        name="innv4_tables",
    )(W1.reshape(1, HID), b1.reshape(1, HID), W2, b2.reshape(1, HID),
      W3.reshape(1, HID), b3.reshape(1, 1))

    tab4 = jnp.broadcast_to(pk.reshape(4, 1, COLS), (4, CH, COLS))
    cvec = cst.reshape(4)

    part = pl.pallas_call(
        _reduce_kernel,
        out_shape=jax.ShapeDtypeStruct((2, CH, COLS), jnp.float32),
        grid=(NBLK,),
        in_specs=[
            pl.BlockSpec(memory_space=pltpu.SMEM),
            pl.BlockSpec((4, CH, COLS), lambda i: (0, 0, 0)),
            pl.BlockSpec((BR, COLS), lambda i: (i, 0)),
        ],
        out_specs=pl.BlockSpec((2, CH, COLS), lambda i: (0, 0, 0)),
        compiler_params=pltpu.CompilerParams(
            dimension_semantics=("arbitrary",)),
        name="innv4_reduce",
    )(cvec, tab4, zr)

    s1 = jnp.sum(part[0])
    s2 = jnp.sum(part[1])
    nb = jnp.float32(B)
    mu = s1 / nb
    var = (s2 - s1 * s1 / nb) / (nb - 1.0)
    sig = jnp.maximum(jnp.sqrt(jnp.maximum(var, 0.0)), 1e-6)
    a = jax.nn.softplus(a_raw) + 1e-3
    s = a / sig
    c = b_param - mu * s
    cvec2 = jnp.stack([s * cvec[0] + c, s * cvec[1],
                       s * cvec[2], s * cvec[3]]).astype(jnp.float32)

    out = pl.pallas_call(
        _apply_kernel,
        out_shape=jax.ShapeDtypeStruct((ROWS, COLS), jnp.float32),
        grid=(NBLK,),
        in_specs=[
            pl.BlockSpec(memory_space=pltpu.SMEM),
            pl.BlockSpec((4, CH, COLS), lambda i: (0, 0, 0)),
            pl.BlockSpec((BR, COLS), lambda i: (i, 0)),
        ],
        out_specs=pl.BlockSpec((BR, COLS), lambda i: (i, 0)),
        compiler_params=pltpu.CompilerParams(
            dimension_semantics=("arbitrary",)),
        name="innv4_apply",
    )(cvec2, tab4, zr)

    return out.reshape(B, 1)


# speedup vs baseline: 1743.0148x; 1.4470x over previous
import jax
import jax.numpy as jnp
from jax.experimental import pallas as pl
from jax.experimental.pallas import tpu as pltpu

NPTS = 400
ZMIN, ZMAX = -3.0, 3.0
CLIP = 1.0
HID = 64
PPAD = 512
DT = (ZMAX - ZMIN) / (NPTS - 1)
INV_DT = 1.0 / DT

B = 8388608
COLS = 128
ROWS = B // COLS
BR = 4096
NBLK = ROWS // BR
CH = 8


def _tables_kernel(w1_ref, b1_ref, w2_ref, b2_ref, w3_ref, b3_ref,
                   pk_ref, cst_ref):
    tf = ZMIN + DT * jax.lax.broadcasted_iota(
        jnp.int32, (PPAD, HID), 0).astype(jnp.float32)
    h = jnp.tanh(tf * w1_ref[...] + b1_ref[...])
    h = jnp.tanh(
        jax.lax.dot_general(h, w2_ref[...], (((1,), (1,)), ((), ())),
                            preferred_element_type=jnp.float32)
        + b2_ref[...])
    g = jnp.sum(h * w3_ref[...], axis=1, keepdims=True) + b3_ref[...]
    w = jnp.exp(jnp.clip(g, -CLIP, CLIP))
    ir = jax.lax.broadcasted_iota(jnp.int32, (PPAD, PPAD), 0)
    jc = jax.lax.broadcasted_iota(jnp.int32, (PPAD, PPAD), 1)
    coeff = ((jc < ir).astype(jnp.float32)
             + ((jc >= 1) & (jc <= ir)).astype(jnp.float32)) * (0.5 * DT)
    F = jax.lax.dot_general(coeff, w, (((1,), (0,)), ((), ())),
                            preferred_element_type=jnp.float32)
    icol = jax.lax.broadcasted_iota(jnp.int32, (PPAD, 1), 0)
    tcol = ZMIN + DT * icol.astype(jnp.float32)
    valid = icol < NPTS
    alpha = jnp.where(valid, F - tcol * w, 0.0)
    beta = jnp.where(valid, w, 1.0)
    amin = jnp.min(alpha, keepdims=True)
    arange = jnp.max(alpha, keepdims=True) - amin
    sa = jnp.maximum(arange * (1.0 / 65535.0), 1e-30)
    bmin = jnp.min(beta, keepdims=True)
    brange = jnp.max(beta, keepdims=True) - bmin
    sb = jnp.maximum(brange * (1.0 / 65535.0), 1e-30)
    qa = ((alpha - amin) * (1.0 / sa) + 0.5).astype(jnp.int32) - 32768
    qb = ((beta - bmin) * (1.0 / sb) + 0.5).astype(jnp.int32)
    pk_ref[...] = jnp.left_shift(qa, 16) | qb
    amin_adj = amin + 32768.0 * sa
    cst_ref[...] = jnp.concatenate([amin_adj, sa, bmin, sb], axis=1)


def _interp_chunk(trows, c_amin, c_sa, c_bmin, c_sb, z):
    pos = (z - ZMIN) * INV_DT
    k = jnp.clip(pos, 0.0, float(NPTS - 2)).astype(jnp.int32)
    k = jnp.where(z > ZMAX, NPTS - 1, k)
    lo = jnp.bitwise_and(k, 127)
    hi = jnp.right_shift(k, 7)
    low = hi < 2
    is0 = hi == 0
    is2 = hi == 2

    def gather(row):
        return jnp.take_along_axis(trows[row], lo, axis=1)

    v01 = jnp.where(is0, gather(0), gather(1))
    v23 = jnp.where(is2, gather(2), gather(3))
    v = jnp.where(low, v01, v23)
    qa = jnp.right_shift(v, 16).astype(jnp.float32)
    qb = jnp.bitwise_and(v, 0xFFFF).astype(jnp.float32)
    t2 = qa * c_sa + c_amin
    t1 = qb * c_sb + c_bmin
    return t1 * z + t2


def _reduce_kernel(cst_ref, tab_ref, z_ref, fz_ref, out_ref):
    i = pl.program_id(0)
    trows = [tab_ref[r] for r in range(4)]
    c_amin, c_sa, c_bmin, c_sb = (cst_ref[0], cst_ref[1],
                                  cst_ref[2], cst_ref[3])
    acc = [jnp.zeros((CH, COLS), jnp.float32) for _ in range(4)]
    for j in range(BR // CH):
        zc = z_ref[pl.ds(j * CH, CH), :]
        fz = _interp_chunk(trows, c_amin, c_sa, c_bmin, c_sb, zc)
        fz_ref[pl.ds(j * CH, CH), :] = fz
        p = j & 1
        acc[p] = acc[p] + fz
        acc[2 + p] = acc[2 + p] + fz * fz

    @pl.when(i == 0)
    def _():
        out_ref[...] = jnp.zeros_like(out_ref)

    out_ref[0] += acc[0] + acc[1]
    out_ref[1] += acc[2] + acc[3]


def _apply_kernel(sc_ref, fz_ref, o_ref):
    o_ref[...] = fz_ref[...] * sc_ref[0] + sc_ref[1]


def kernel(z, W1, b1, W2, b2, W3, b3, a_raw, b_param):
    zr = z.reshape(ROWS, COLS)

    pk, cst = pl.pallas_call(
        _tables_kernel,
        out_shape=(jax.ShapeDtypeStruct((PPAD, 1), jnp.int32),
                   jax.ShapeDtypeStruct((1, 4), jnp.float32)),
        name="innv4_tables",
    )(W1.reshape(1, HID), b1.reshape(1, HID), W2, b2.reshape(1, HID),
      W3.reshape(1, HID), b3.reshape(1, 1))

    tab4 = jnp.broadcast_to(pk.reshape(4, 1, COLS), (4, CH, COLS))
    cvec = cst.reshape(4)

    fzr, part = pl.pallas_call(
        _reduce_kernel,
        out_shape=(jax.ShapeDtypeStruct((ROWS, COLS), jnp.float32),
                   jax.ShapeDtypeStruct((2, CH, COLS), jnp.float32)),
        grid=(NBLK,),
        in_specs=[
            pl.BlockSpec(memory_space=pltpu.SMEM),
            pl.BlockSpec((4, CH, COLS), lambda i: (0, 0, 0)),
            pl.BlockSpec((BR, COLS), lambda i: (i, 0)),
        ],
        out_specs=(pl.BlockSpec((BR, COLS), lambda i: (i, 0)),
                   pl.BlockSpec((2, CH, COLS), lambda i: (0, 0, 0))),
        compiler_params=pltpu.CompilerParams(
            dimension_semantics=("arbitrary",)),
        name="innv4_reduce",
    )(cvec, tab4, zr)

    s1 = jnp.sum(part[0])
    s2 = jnp.sum(part[1])
    nb = jnp.float32(B)
    mu = s1 / nb
    var = (s2 - s1 * s1 / nb) / (nb - 1.0)
    sig = jnp.maximum(jnp.sqrt(jnp.maximum(var, 0.0)), 1e-6)
    a = jax.nn.softplus(a_raw) + 1e-3
    s = a / sig
    c = b_param - mu * s
    sc = jnp.stack([s, c]).astype(jnp.float32)

    out = pl.pallas_call(
        _apply_kernel,
        out_shape=jax.ShapeDtypeStruct((ROWS, COLS), jnp.float32),
        grid=(NBLK,),
        in_specs=[
            pl.BlockSpec(memory_space=pltpu.SMEM),
            pl.BlockSpec((BR, COLS), lambda i: (i, 0)),
        ],
        out_specs=pl.BlockSpec((BR, COLS), lambda i: (i, 0)),
        compiler_params=pltpu.CompilerParams(
            dimension_semantics=("arbitrary",)),
        name="innv4_apply",
    )(sc, fzr)

    return out.reshape(B, 1)


# speedup vs baseline: 2092.2497x; 1.2004x over previous
import jax
import jax.numpy as jnp
from jax.experimental import pallas as pl
from jax.experimental.pallas import tpu as pltpu

NPTS = 400
ZMIN, ZMAX = -3.0, 3.0
CLIP = 1.0
HID = 64
PPAD = 512
DT = (ZMAX - ZMIN) / (NPTS - 1)
INV_DT = 1.0 / DT

B = 8388608
COLS = 128
ROWS = B // COLS
BR = 4096
NBLK = ROWS // BR
CH = 8


def _tables_kernel(w1_ref, b1_ref, w2_ref, b2_ref, w3_ref, b3_ref,
                   pk_ref, cst_ref):
    tf = ZMIN + DT * jax.lax.broadcasted_iota(
        jnp.int32, (PPAD, HID), 0).astype(jnp.float32)
    h = jnp.tanh(tf * w1_ref[...] + b1_ref[...])
    h = jnp.tanh(
        jax.lax.dot_general(h, w2_ref[...], (((1,), (1,)), ((), ())),
                            preferred_element_type=jnp.float32)
        + b2_ref[...])
    g = jnp.sum(h * w3_ref[...], axis=1, keepdims=True) + b3_ref[...]
    w = jnp.exp(jnp.clip(g, -CLIP, CLIP))
    ir = jax.lax.broadcasted_iota(jnp.int32, (PPAD, PPAD), 0)
    jc = jax.lax.broadcasted_iota(jnp.int32, (PPAD, PPAD), 1)
    coeff = ((jc < ir).astype(jnp.float32)
             + ((jc >= 1) & (jc <= ir)).astype(jnp.float32)) * (0.5 * DT)
    F = jax.lax.dot_general(coeff, w, (((1,), (0,)), ((), ())),
                            preferred_element_type=jnp.float32)
    icol = jax.lax.broadcasted_iota(jnp.int32, (PPAD, 1), 0)
    tcol = ZMIN + DT * icol.astype(jnp.float32)
    valid = icol < NPTS
    alpha = jnp.where(valid, F - tcol * w, 0.0)
    beta = jnp.where(valid, w, 1.0)
    amin = jnp.min(alpha, keepdims=True)
    arange = jnp.max(alpha, keepdims=True) - amin
    sa = jnp.maximum(arange * (1.0 / 65535.0), 1e-30)
    bmin = jnp.min(beta, keepdims=True)
    brange = jnp.max(beta, keepdims=True) - bmin
    sb = jnp.maximum(brange * (1.0 / 65535.0), 1e-30)
    qa = ((alpha - amin) * (1.0 / sa) + 0.5).astype(jnp.int32) - 32768
    qb = ((beta - bmin) * (1.0 / sb) + 0.5).astype(jnp.int32)
    pk_ref[...] = jnp.left_shift(qa, 16) | qb
    amin_adj = amin + 32768.0 * sa
    cst_ref[...] = jnp.concatenate([amin_adj, sa, bmin, sb], axis=1)


def _interp_chunk(trows, c_amin, c_sa, c_bmin, c_sb, z):
    pos = (z - ZMIN) * INV_DT
    k = jnp.clip(pos, 0.0, float(NPTS - 2)).astype(jnp.int32)
    k = jnp.where(z > ZMAX, NPTS - 1, k)
    lo = jnp.bitwise_and(k, 127)
    hi = jnp.right_shift(k, 7)
    low = hi < 2
    is0 = hi == 0
    is2 = hi == 2

    def gather(row):
        return jnp.take_along_axis(trows[row], lo, axis=1)

    v01 = jnp.where(is0, gather(0), gather(1))
    v23 = jnp.where(is2, gather(2), gather(3))
    v = jnp.where(low, v01, v23)
    qa = jnp.right_shift(v, 16).astype(jnp.float32)
    qb = jnp.bitwise_and(v, 0xFFFF).astype(jnp.float32)
    t2 = qa * c_sa + c_amin
    t1 = qb * c_sb + c_bmin
    return t1 * z + t2


def _main_kernel(cst_ref, ab_ref, tab_ref, z_ref, o_ref,
                 fz_ref, acc_ref, sc_ref):
    p = pl.program_id(0)
    i = pl.program_id(1)

    @pl.when((p == 0) & (i == 0))
    def _():
        acc_ref[...] = jnp.zeros_like(acc_ref)

    @pl.when(p == 0)
    def _():
        trows = [tab_ref[r] for r in range(4)]
        c_amin, c_sa, c_bmin, c_sb = (cst_ref[0], cst_ref[1],
                                      cst_ref[2], cst_ref[3])
        acc = [jnp.zeros((CH, COLS), jnp.float32) for _ in range(4)]
        for j in range(BR // CH):
            zc = z_ref[pl.ds(j * CH, CH), :]
            fz = _interp_chunk(trows, c_amin, c_sa, c_bmin, c_sb, zc)
            fz_ref[i, pl.ds(j * CH, CH), :] = fz
            q = j & 1
            acc[q] = acc[q] + fz
            acc[2 + q] = acc[2 + q] + fz * fz
        acc_ref[0] += acc[0] + acc[1]
        acc_ref[1] += acc[2] + acc[3]

    @pl.when((p == 1) & (i == 0))
    def _():
        nb = jnp.float32(B)
        s1 = jnp.sum(acc_ref[0])
        s2 = jnp.sum(acc_ref[1])
        mu = s1 / nb
        var = (s2 - s1 * s1 / nb) / (nb - 1.0)
        sig = jnp.maximum(jnp.sqrt(jnp.maximum(var, 0.0)), 1e-6)
        a = jax.nn.softplus(ab_ref[0]) + 1e-3
        s = a / sig
        sc_ref[0] = s
        sc_ref[1] = ab_ref[1] - mu * s

    @pl.when(p == 1)
    def _():
        o_ref[...] = fz_ref[i] * sc_ref[0] + sc_ref[1]


def kernel(z, W1, b1, W2, b2, W3, b3, a_raw, b_param):
    zr = z.reshape(ROWS, COLS)

    pk, cst = pl.pallas_call(
        _tables_kernel,
        out_shape=(jax.ShapeDtypeStruct((PPAD, 1), jnp.int32),
                   jax.ShapeDtypeStruct((1, 4), jnp.float32)),
        name="innv4_tables",
    )(W1.reshape(1, HID), b1.reshape(1, HID), W2, b2.reshape(1, HID),
      W3.reshape(1, HID), b3.reshape(1, 1))

    tab4 = jnp.broadcast_to(pk.reshape(4, 1, COLS), (4, CH, COLS))
    cvec = cst.reshape(4)

    ab2 = jnp.stack([a_raw, b_param]).astype(jnp.float32)

    out = pl.pallas_call(
        _main_kernel,
        out_shape=jax.ShapeDtypeStruct((ROWS, COLS), jnp.float32),
        grid=(2, NBLK),
        in_specs=[
            pl.BlockSpec(memory_space=pltpu.SMEM),
            pl.BlockSpec(memory_space=pltpu.SMEM),
            pl.BlockSpec((4, CH, COLS), lambda p, i: (0, 0, 0)),
            pl.BlockSpec((BR, COLS), lambda p, i: (jnp.where(p == 0, i, 0), 0)),
        ],
        out_specs=pl.BlockSpec(
            (BR, COLS), lambda p, i: (jnp.where(p == 1, i, 0), 0)),
        scratch_shapes=[
            pltpu.VMEM((NBLK, BR, COLS), jnp.float32),
            pltpu.VMEM((2, CH, COLS), jnp.float32),
            pltpu.SMEM((2,), jnp.float32),
        ],
        compiler_params=pltpu.CompilerParams(
            dimension_semantics=("arbitrary", "arbitrary"),
            vmem_limit_bytes=56 * 1024 * 1024),
        name="innv4_main",
    )(cvec, ab2, tab4, zr)

    return out.reshape(B, 1)


# speedup vs baseline: 2160.3481x; 1.0325x over previous
import jax
import jax.numpy as jnp
from jax.experimental import pallas as pl
from jax.experimental.pallas import tpu as pltpu

NPTS = 400
ZMIN, ZMAX = -3.0, 3.0
CLIP = 1.0
HID = 64
PPAD = 512
DT = (ZMAX - ZMIN) / (NPTS - 1)
INV_DT = 1.0 / DT

B = 8388608
COLS = 128
ROWS = B // COLS
BR = 8192
NBLK = ROWS // BR
CH = 8


def _tables_kernel(w1_ref, b1_ref, w2_ref, b2_ref, w3_ref, b3_ref,
                   pk_ref, cst_ref):
    tf = ZMIN + DT * jax.lax.broadcasted_iota(
        jnp.int32, (PPAD, HID), 0).astype(jnp.float32)
    h = jnp.tanh(tf * w1_ref[...] + b1_ref[...])
    h = jnp.tanh(
        jax.lax.dot_general(h, w2_ref[...], (((1,), (1,)), ((), ())),
                            preferred_element_type=jnp.float32)
        + b2_ref[...])
    g = jnp.sum(h * w3_ref[...], axis=1, keepdims=True) + b3_ref[...]
    w = jnp.exp(jnp.clip(g, -CLIP, CLIP))
    ir = jax.lax.broadcasted_iota(jnp.int32, (PPAD, PPAD), 0)
    jc = jax.lax.broadcasted_iota(jnp.int32, (PPAD, PPAD), 1)
    coeff = ((jc < ir).astype(jnp.float32)
             + ((jc >= 1) & (jc <= ir)).astype(jnp.float32)) * (0.5 * DT)
    F = jax.lax.dot_general(coeff, w, (((1,), (0,)), ((), ())),
                            preferred_element_type=jnp.float32)
    icol = jax.lax.broadcasted_iota(jnp.int32, (PPAD, 1), 0)
    tcol = ZMIN + DT * icol.astype(jnp.float32)
    valid = icol < NPTS
    alpha = jnp.where(valid, F - tcol * w, 0.0)
    beta = jnp.where(valid, w, 1.0)
    amin = jnp.min(alpha, keepdims=True)
    arange = jnp.max(alpha, keepdims=True) - amin
    sa = jnp.maximum(arange * (1.0 / 65535.0), 1e-30)
    bmin = jnp.min(beta, keepdims=True)
    brange = jnp.max(beta, keepdims=True) - bmin
    sb = jnp.maximum(brange * (1.0 / 65535.0), 1e-30)
    qa = ((alpha - amin) * (1.0 / sa) + 0.5).astype(jnp.int32) - 32768
    qb = ((beta - bmin) * (1.0 / sb) + 0.5).astype(jnp.int32)
    pk_ref[...] = jnp.left_shift(qa, 16) | qb
    amin_adj = amin + 32768.0 * sa
    cst_ref[...] = jnp.concatenate([amin_adj, sa, bmin, sb], axis=1)


def _interp_chunk(trows, c_amin, c_sa, c_bmin, c_sb, z):
    pos = (z - ZMIN) * INV_DT
    k = jnp.clip(pos, 0.0, float(NPTS - 2)).astype(jnp.int32)
    k = jnp.where(z > ZMAX, NPTS - 1, k)
    lo = jnp.bitwise_and(k, 127)
    hi = jnp.right_shift(k, 7)
    low = hi < 2
    is0 = hi == 0
    is2 = hi == 2

    def gather(row):
        return jnp.take_along_axis(trows[row], lo, axis=1)

    v01 = jnp.where(is0, gather(0), gather(1))
    v23 = jnp.where(is2, gather(2), gather(3))
    v = jnp.where(low, v01, v23)
    qa = jnp.right_shift(v, 16).astype(jnp.float32)
    qb = jnp.bitwise_and(v, 0xFFFF).astype(jnp.float32)
    t2 = qa * c_sa + c_amin
    t1 = qb * c_sb + c_bmin
    return t1 * z + t2


def _main_kernel(cst_ref, ab_ref, tab_ref, z_ref, o_ref,
                 fz_ref, acc_ref, sc_ref):
    p = pl.program_id(0)
    i = pl.program_id(1)

    @pl.when((p == 0) & (i == 0))
    def _():
        acc_ref[...] = jnp.zeros_like(acc_ref)

    @pl.when(p == 0)
    def _():
        trows = [tab_ref[r] for r in range(4)]
        c_amin, c_sa, c_bmin, c_sb = (cst_ref[0], cst_ref[1],
                                      cst_ref[2], cst_ref[3])
        acc = [jnp.zeros((CH, COLS), jnp.float32) for _ in range(4)]
        for j in range(BR // CH):
            zc = z_ref[pl.ds(j * CH, CH), :]
            fz = _interp_chunk(trows, c_amin, c_sa, c_bmin, c_sb, zc)
            fz_ref[i, pl.ds(j * CH, CH), :] = fz
            q = j & 1
            acc[q] = acc[q] + fz
            acc[2 + q] = acc[2 + q] + fz * fz
        acc_ref[0] += acc[0] + acc[1]
        acc_ref[1] += acc[2] + acc[3]

    @pl.when((p == 1) & (i == 0))
    def _():
        nb = jnp.float32(B)
        s1 = jnp.sum(acc_ref[0])
        s2 = jnp.sum(acc_ref[1])
        mu = s1 / nb
        var = (s2 - s1 * s1 / nb) / (nb - 1.0)
        sig = jnp.maximum(jnp.sqrt(jnp.maximum(var, 0.0)), 1e-6)
        a = jax.nn.softplus(ab_ref[0]) + 1e-3
        s = a / sig
        sc_ref[0] = s
        sc_ref[1] = ab_ref[1] - mu * s

    @pl.when(p == 1)
    def _():
        o_ref[...] = fz_ref[i] * sc_ref[0] + sc_ref[1]


def kernel(z, W1, b1, W2, b2, W3, b3, a_raw, b_param):
    zr = z.reshape(ROWS, COLS)

    pk, cst = pl.pallas_call(
        _tables_kernel,
        out_shape=(jax.ShapeDtypeStruct((PPAD, 1), jnp.int32),
                   jax.ShapeDtypeStruct((1, 4), jnp.float32)),
        name="innv4_tables",
    )(W1.reshape(1, HID), b1.reshape(1, HID), W2, b2.reshape(1, HID),
      W3.reshape(1, HID), b3.reshape(1, 1))

    tab4 = jnp.broadcast_to(pk.reshape(4, 1, COLS), (4, CH, COLS))
    cvec = cst.reshape(4)

    ab2 = jnp.stack([a_raw, b_param]).astype(jnp.float32)

    out = pl.pallas_call(
        _main_kernel,
        out_shape=jax.ShapeDtypeStruct((ROWS, COLS), jnp.float32),
        grid=(2, NBLK),
        in_specs=[
            pl.BlockSpec(memory_space=pltpu.SMEM),
            pl.BlockSpec(memory_space=pltpu.SMEM),
            pl.BlockSpec((4, CH, COLS), lambda p, i: (0, 0, 0)),
            pl.BlockSpec((BR, COLS), lambda p, i: (jnp.where(p == 0, i, 0), 0)),
        ],
        out_specs=pl.BlockSpec(
            (BR, COLS), lambda p, i: (jnp.where(p == 1, i, 0), 0)),
        scratch_shapes=[
            pltpu.VMEM((NBLK, BR, COLS), jnp.float32),
            pltpu.VMEM((2, CH, COLS), jnp.float32),
            pltpu.SMEM((2,), jnp.float32),
        ],
        compiler_params=pltpu.CompilerParams(
            dimension_semantics=("arbitrary", "arbitrary"),
            vmem_limit_bytes=56 * 1024 * 1024),
        name="innv4_main",
    )(cvec, ab2, tab4, zr)

    return out.reshape(B, 1)
